# Initial kernel scaffold; baseline (speedup 1.0000x reference)
import functools

import jax
import jax.numpy as jnp
from jax import lax
from jax.experimental import pallas as pl
from jax.experimental.pallas import tpu as pltpu
from jax.experimental.pallas import tpu_sc as plsc

_N = 10000
_E = 320000
_D = 128
_H = 32
_C = 40

_NC = 2
_NS = 16
_NW = _NC * _NS
_EPW = _E // _NW
_B = 80
_G = _EPW // _B
_RPS = _N // _NS
_ZR = 125
_DW = 16


def _agg_body(with_deg, y_hbm, src_hbm, dst_hbm, *refs):
    if with_deg:
        (acc_out, deg_out, src_v, dst_v, rows_v, zb_v, acc_sp, sem,
         ones_v, zd_v, deg_sp) = refs
    else:
        (acc_out, src_v, dst_v, rows_v, zb_v, acc_sp, sem) = refs
    c = lax.axis_index("c")
    s = lax.axis_index("s")
    wid = c * _NS + s

    pltpu.sync_copy(src_hbm.at[pl.ds(wid * _G, _G)], src_v)
    pltpu.sync_copy(dst_hbm.at[pl.ds(wid * _G, _G)], dst_v)

    def zf(i, _):
        zb_v[i, pl.ds(0, 16)] = jnp.zeros((16,), jnp.float32)
        zb_v[i, pl.ds(16, 16)] = jnp.zeros((16,), jnp.float32)
        return 0

    lax.fori_loop(0, _ZR, zf, 0)
    row0 = s * _RPS
    for k in range(_RPS // _ZR):
        pltpu.sync_copy(zb_v, acc_sp.at[pl.ds(row0 + k * _ZR, _ZR)])
    if with_deg:
        def zf2(i, _):
            zd_v[i, :] = jnp.zeros((_DW,), jnp.float32)
            return 0

        lax.fori_loop(0, _ZR, zf2, 0)

        def of(i, _):
            ones_v[i, :] = jnp.ones((_DW,), jnp.float32)
            return 0

        lax.fori_loop(0, _B, of, 0)
        for k in range(_RPS // _ZR):
            pltpu.sync_copy(zd_v, deg_sp.at[pl.ds(row0 + k * _ZR, _ZR)])

    plsc.subcore_barrier()

    def step(j, _):
        pltpu.async_copy(y_hbm.at[src_v.at[j]], rows_v, sem).wait()
        pltpu.sync_copy(rows_v, acc_sp.at[dst_v.at[j]], add=True)
        if with_deg:
            pltpu.sync_copy(ones_v, deg_sp.at[dst_v.at[j]], add=True)
        return 0

    lax.fori_loop(0, _G, step, 0)
    plsc.subcore_barrier()

    pltpu.sync_copy(acc_sp.at[pl.ds(row0, _RPS)], acc_out.at[c, pl.ds(row0, _RPS)])
    if with_deg:
        pltpu.sync_copy(deg_sp.at[pl.ds(row0, _RPS)], deg_out.at[c, pl.ds(row0, _RPS)])


def _make_agg(with_deg):
    mesh = plsc.VectorSubcoreMesh(core_axis_name="c", subcore_axis_name="s")
    out_type = [jax.ShapeDtypeStruct((_NC, _N, _H), jnp.float32)]
    scratch = [
        pltpu.VMEM((_G, _B), jnp.int32),
        pltpu.VMEM((_G, _B), jnp.int32),
        pltpu.VMEM((_B, _H), jnp.float32),
        pltpu.VMEM((_ZR, _H), jnp.float32),
        pltpu.VMEM_SHARED((_N, _H), jnp.float32),
        pltpu.SemaphoreType.DMA,
    ]
    if with_deg:
        out_type.append(jax.ShapeDtypeStruct((_NC, _N, _DW), jnp.float32))
        scratch += [
            pltpu.VMEM((_B, _DW), jnp.float32),
            pltpu.VMEM((_ZR, _DW), jnp.float32),
            pltpu.VMEM_SHARED((_N, _DW), jnp.float32),
        ]
    return pl.kernel(
        functools.partial(_agg_body, with_deg),
        out_type=out_type,
        mesh=mesh,
        scratch_types=scratch,
    )


_agg_deg = _make_agg(True)
_agg = _make_agg(False)


def _proj1_body(x_ref, wl_ref, wr_ref, bl_ref, y_ref, z_ref):
    xb = x_ref[...]
    dn = (((1,), (1,)), ((), ()))
    y_ref[...] = lax.dot_general(xb, wl_ref[...], dn,
                                 preferred_element_type=jnp.float32)
    z_ref[...] = lax.dot_general(xb, wr_ref[...], dn,
                                 preferred_element_type=jnp.float32) + bl_ref[...]


def _proj1(x, wl1, wr1, bl1):
    bn = 2000
    return pl.pallas_call(
        _proj1_body,
        grid=(_N // bn,),
        in_specs=[
            pl.BlockSpec((bn, _D), lambda i: (i, 0)),
            pl.BlockSpec((_H, _D), lambda i: (0, 0)),
            pl.BlockSpec((_H, _D), lambda i: (0, 0)),
            pl.BlockSpec((1, _H), lambda i: (0, 0)),
        ],
        out_specs=[
            pl.BlockSpec((bn, _H), lambda i: (i, 0)),
            pl.BlockSpec((bn, _H), lambda i: (i, 0)),
        ],
        out_shape=[
            jax.ShapeDtypeStruct((_N, _H), jnp.float32),
            jax.ShapeDtypeStruct((_N, _H), jnp.float32),
        ],
    )(x, wl1, wr1, bl1)


def _mid_body(s_ref, d_ref, z_ref, h_ref, dinv_ref):
    ssum = s_ref[0] + s_ref[1]
    deg = d_ref[0, :, 0:1] + d_ref[1, :, 0:1]
    dinv = 1.0 / jnp.maximum(deg, 1.0)
    h_ref[...] = jnp.maximum(ssum * dinv + z_ref[...], 0.0)
    dinv_ref[...] = dinv


def _mid(s1, deg, z1):
    bn = 2000
    return pl.pallas_call(
        _mid_body,
        grid=(_N // bn,),
        in_specs=[
            pl.BlockSpec((_NC, bn, _H), lambda i: (0, i, 0)),
            pl.BlockSpec((_NC, bn, _DW), lambda i: (0, i, 0)),
            pl.BlockSpec((bn, _H), lambda i: (i, 0)),
        ],
        out_specs=[
            pl.BlockSpec((bn, _H), lambda i: (i, 0)),
            pl.BlockSpec((bn, 1), lambda i: (i, 0)),
        ],
        out_shape=[
            jax.ShapeDtypeStruct((_N, _H), jnp.float32),
            jax.ShapeDtypeStruct((_N, 1), jnp.float32),
        ],
    )(s1, deg, z1)


def _final_body(s2_ref, dinv_ref, h_ref, wl2_ref, bl2_ref, wr2_ref, o_ref):
    mean2 = (s2_ref[0] + s2_ref[1]) * dinv_ref[...]
    dn = (((1,), (1,)), ((), ()))
    t = (lax.dot_general(mean2, wl2_ref[...], dn,
                         preferred_element_type=jnp.float32)
         + bl2_ref[...]
         + lax.dot_general(h_ref[...], wr2_ref[...], dn,
                           preferred_element_type=jnp.float32))
    m = jnp.max(t, axis=1, keepdims=True)
    lse = jnp.log(jnp.sum(jnp.exp(t - m), axis=1, keepdims=True))
    o_ref[...] = t - m - lse


def _final(s2, dinv, h, wl2, bl2, wr2):
    bn = 2000
    return pl.pallas_call(
        _final_body,
        grid=(_N // bn,),
        in_specs=[
            pl.BlockSpec((_NC, bn, _H), lambda i: (0, i, 0)),
            pl.BlockSpec((bn, 1), lambda i: (i, 0)),
            pl.BlockSpec((bn, _H), lambda i: (i, 0)),
            pl.BlockSpec((_C, _H), lambda i: (0, 0)),
            pl.BlockSpec((1, _C), lambda i: (0, 0)),
            pl.BlockSpec((_C, _H), lambda i: (0, 0)),
        ],
        out_specs=pl.BlockSpec((bn, _C), lambda i: (i, 0)),
        out_shape=jax.ShapeDtypeStruct((_N, _C), jnp.float32),
    )(s2, dinv, h, wl2, bl2, wr2)


def kernel(x, edge_index, Wl1, bl1, Wr1, Wl2, bl2, Wr2):
    src = edge_index[0].reshape(_NW * _G, _B)
    dst = edge_index[1].reshape(_NW * _G, _B)
    y1, z1 = _proj1(x, Wl1, Wr1, bl1.reshape(1, _H))
    s1, deg = _agg_deg(y1, src, dst)
    h, dinv = _mid(s1, deg, z1)
    s2 = _agg(h, src, dst)
    if isinstance(s2, (list, tuple)):
        s2 = s2[0]
    out = _final(s2, dinv, h, Wl2, bl2.reshape(1, _C), Wr2)
    return out


# speedup vs baseline: 10.8750x; 10.8750x over previous
import functools

import jax
import jax.numpy as jnp
from jax import lax
from jax.experimental import pallas as pl
from jax.experimental.pallas import tpu as pltpu
from jax.experimental.pallas import tpu_sc as plsc

_N = 10000
_E = 320000
_D = 128
_H = 32
_C = 40

_NC = 2
_NS = 16
_NW = _NC * _NS
_EPW = _E // _NW
_B = 80
_G = _EPW // _B
_RPS = _N // _NS
_ZR = 125
_DW = 16


def _agg_body(with_deg, y_hbm, src_hbm, dst_hbm, *refs):
    if with_deg:
        (acc_out, deg_out, src_v, dst_v, rows_v, zb_v, acc_sp, sem,
         ones_v, zd_v, deg_sp) = refs
    else:
        (acc_out, src_v, dst_v, rows_v, zb_v, acc_sp, sem) = refs
    c = lax.axis_index("c")
    s = lax.axis_index("s")
    wid = c * _NS + s

    pltpu.sync_copy(src_hbm.at[wid], src_v)
    pltpu.sync_copy(dst_hbm.at[wid], dst_v)

    def zf(i, _):
        zb_v[i, pl.ds(0, 16)] = jnp.zeros((16,), jnp.float32)
        zb_v[i, pl.ds(16, 16)] = jnp.zeros((16,), jnp.float32)
        return 0

    lax.fori_loop(0, _ZR, zf, 0)
    row0 = s * _RPS
    for k in range(_RPS // _ZR):
        pltpu.sync_copy(zb_v, acc_sp.at[pl.ds(row0 + k * _ZR, _ZR)])
    if with_deg:
        def zf2(i, _):
            zd_v[i, :] = jnp.zeros((_DW,), jnp.float32)
            return 0

        lax.fori_loop(0, _ZR, zf2, 0)

        def of(i, _):
            ones_v[i, :] = jnp.ones((_DW,), jnp.float32)
            return 0

        lax.fori_loop(0, _B, of, 0)
        for k in range(_RPS // _ZR):
            pltpu.sync_copy(zd_v, deg_sp.at[pl.ds(row0 + k * _ZR, _ZR)])

    plsc.subcore_barrier()

    def step(j, _):
        pltpu.async_copy(y_hbm.at[src_v.at[j]], rows_v, sem).wait()
        pltpu.sync_copy(rows_v, acc_sp.at[dst_v.at[j]], add=True)
        if with_deg:
            pltpu.sync_copy(ones_v, deg_sp.at[dst_v.at[j]], add=True)
        return 0

    lax.fori_loop(0, _G, step, 0)
    plsc.subcore_barrier()

    pltpu.sync_copy(acc_sp.at[pl.ds(row0, _RPS)], acc_out.at[c, s])
    if with_deg:
        pltpu.sync_copy(deg_sp.at[pl.ds(row0, _RPS)], deg_out.at[c, s])


def _make_agg(with_deg):
    mesh = plsc.VectorSubcoreMesh(core_axis_name="c", subcore_axis_name="s")
    out_type = [jax.ShapeDtypeStruct((_NC, _NS, _RPS, _H), jnp.float32)]
    scratch = [
        pltpu.VMEM((_G, _B), jnp.int32),
        pltpu.VMEM((_G, _B), jnp.int32),
        pltpu.VMEM((_B, _H), jnp.float32),
        pltpu.VMEM((_ZR, _H), jnp.float32),
        pltpu.VMEM_SHARED((_N, _H), jnp.float32),
        pltpu.SemaphoreType.DMA,
    ]
    if with_deg:
        out_type.append(jax.ShapeDtypeStruct((_NC, _NS, _RPS, _DW), jnp.float32))
        scratch += [
            pltpu.VMEM((_B, _DW), jnp.float32),
            pltpu.VMEM((_ZR, _DW), jnp.float32),
            pltpu.VMEM_SHARED((_N, _DW), jnp.float32),
        ]
    return pl.kernel(
        functools.partial(_agg_body, with_deg),
        out_type=out_type,
        mesh=mesh,
        scratch_types=scratch,
        compiler_params=pltpu.CompilerParams(use_tc_tiling_on_sc=False),
    )


_agg_deg = _make_agg(True)
_agg = _make_agg(False)


def _proj1_body(x_ref, wl_ref, wr_ref, bl_ref, y_ref, z_ref):
    xb = x_ref[...]
    dn = (((1,), (1,)), ((), ()))
    y_ref[...] = lax.dot_general(xb, wl_ref[...], dn,
                                 preferred_element_type=jnp.float32)
    z_ref[...] = lax.dot_general(xb, wr_ref[...], dn,
                                 preferred_element_type=jnp.float32) + bl_ref[...]


def _proj1(x, wl1, wr1, bl1):
    bn = 2000
    return pl.pallas_call(
        _proj1_body,
        grid=(_N // bn,),
        in_specs=[
            pl.BlockSpec((bn, _D), lambda i: (i, 0)),
            pl.BlockSpec((_H, _D), lambda i: (0, 0)),
            pl.BlockSpec((_H, _D), lambda i: (0, 0)),
            pl.BlockSpec((1, _H), lambda i: (0, 0)),
        ],
        out_specs=[
            pl.BlockSpec((bn, _H), lambda i: (i, 0)),
            pl.BlockSpec((bn, _H), lambda i: (i, 0)),
        ],
        out_shape=[
            jax.ShapeDtypeStruct((_N, _H), jnp.float32),
            jax.ShapeDtypeStruct((_N, _H), jnp.float32),
        ],
    )(x, wl1, wr1, bl1)


def _mid_body(s_ref, d_ref, z_ref, h_ref, dinv_ref):
    ssum = s_ref[0] + s_ref[1]
    deg = d_ref[0, :, 0:1] + d_ref[1, :, 0:1]
    dinv = 1.0 / jnp.maximum(deg, 1.0)
    h_ref[...] = jnp.maximum(ssum * dinv + z_ref[...], 0.0)
    dinv_ref[...] = dinv


def _mid(s1, deg, z1):
    bn = 2000
    return pl.pallas_call(
        _mid_body,
        grid=(_N // bn,),
        in_specs=[
            pl.BlockSpec((_NC, bn, _H), lambda i: (0, i, 0)),
            pl.BlockSpec((_NC, bn, _DW), lambda i: (0, i, 0)),
            pl.BlockSpec((bn, _H), lambda i: (i, 0)),
        ],
        out_specs=[
            pl.BlockSpec((bn, _H), lambda i: (i, 0)),
            pl.BlockSpec((bn, 1), lambda i: (i, 0)),
        ],
        out_shape=[
            jax.ShapeDtypeStruct((_N, _H), jnp.float32),
            jax.ShapeDtypeStruct((_N, 1), jnp.float32),
        ],
    )(s1, deg, z1)


def _final_body(s2_ref, dinv_ref, h_ref, wl2_ref, bl2_ref, wr2_ref, o_ref):
    mean2 = (s2_ref[0] + s2_ref[1]) * dinv_ref[...]
    dn = (((1,), (1,)), ((), ()))
    t = (lax.dot_general(mean2, wl2_ref[...], dn,
                         preferred_element_type=jnp.float32)
         + bl2_ref[...]
         + lax.dot_general(h_ref[...], wr2_ref[...], dn,
                           preferred_element_type=jnp.float32))
    m = jnp.max(t, axis=1, keepdims=True)
    lse = jnp.log(jnp.sum(jnp.exp(t - m), axis=1, keepdims=True))
    o_ref[...] = t - m - lse


def _final(s2, dinv, h, wl2, bl2, wr2):
    bn = 2000
    return pl.pallas_call(
        _final_body,
        grid=(_N // bn,),
        in_specs=[
            pl.BlockSpec((_NC, bn, _H), lambda i: (0, i, 0)),
            pl.BlockSpec((bn, 1), lambda i: (i, 0)),
            pl.BlockSpec((bn, _H), lambda i: (i, 0)),
            pl.BlockSpec((_C, _H), lambda i: (0, 0)),
            pl.BlockSpec((1, _C), lambda i: (0, 0)),
            pl.BlockSpec((_C, _H), lambda i: (0, 0)),
        ],
        out_specs=pl.BlockSpec((bn, _C), lambda i: (i, 0)),
        out_shape=jax.ShapeDtypeStruct((_N, _C), jnp.float32),
    )(s2, dinv, h, wl2, bl2, wr2)


def kernel(x, edge_index, Wl1, bl1, Wr1, Wl2, bl2, Wr2):
    src = edge_index[0].reshape(_NW, _G, _B)
    dst = edge_index[1].reshape(_NW, _G, _B)
    y1, z1 = _proj1(x, Wl1, Wr1, bl1.reshape(1, _H))
    s1, deg = _agg_deg(y1, src, dst)
    s1 = s1.reshape(_NC, _N, _H)
    deg = deg.reshape(_NC, _N, _DW)
    h, dinv = _mid(s1, deg, z1)
    s2 = _agg(h, src, dst)
    if isinstance(s2, (list, tuple)):
        s2 = s2[0]
    s2 = s2.reshape(_NC, _N, _H)
    out = _final(s2, dinv, h, Wl2, bl2.reshape(1, _C), Wr2)
    return out


# speedup vs baseline: 16.5899x; 1.5255x over previous
import functools

import jax
import jax.numpy as jnp
from jax import lax
from jax.experimental import pallas as pl
from jax.experimental.pallas import tpu as pltpu
from jax.experimental.pallas import tpu_sc as plsc

_N = 10000
_E = 320000
_D = 128
_H = 32
_C = 40

_NC = 2
_NS = 16
_NW = _NC * _NS
_EPW = _E // _NW
_B = 100
_G = _EPW // _B
_RPS = _N // _NS
_ZR = 125
_DW = 16


def _agg_body(with_deg, y_hbm, src_hbm, dst_hbm, *refs):
    if with_deg:
        (acc_out, deg_out, src_v, dst_v, rows0_v, rows1_v, zb_v, acc_sp,
         sem0, sem1, ones_v, zd_v, deg_sp) = refs
    else:
        (acc_out, src_v, dst_v, rows0_v, rows1_v, zb_v, acc_sp,
         sem0, sem1) = refs
    c = lax.axis_index("c")
    s = lax.axis_index("s")
    wid = c * _NS + s

    pltpu.sync_copy(src_hbm.at[wid], src_v)
    pltpu.sync_copy(dst_hbm.at[wid], dst_v)

    def zf(i, _):
        zb_v[i, pl.ds(0, 16)] = jnp.zeros((16,), jnp.float32)
        zb_v[i, pl.ds(16, 16)] = jnp.zeros((16,), jnp.float32)
        return 0

    lax.fori_loop(0, _ZR, zf, 0)
    row0 = s * _RPS
    for k in range(_RPS // _ZR):
        pltpu.sync_copy(zb_v, acc_sp.at[pl.ds(row0 + k * _ZR, _ZR)])
    if with_deg:
        def zf2(i, _):
            zd_v[i, :] = jnp.zeros((_DW,), jnp.float32)
            return 0

        lax.fori_loop(0, _ZR, zf2, 0)

        def of(i, _):
            ones_v[i, :] = jnp.ones((_DW,), jnp.float32)
            return 0

        lax.fori_loop(0, _B, of, 0)
        for k in range(_RPS // _ZR):
            pltpu.sync_copy(zd_v, deg_sp.at[pl.ds(row0 + k * _ZR, _ZR)])

    plsc.subcore_barrier()

    pltpu.async_copy(y_hbm.at[src_v.at[0]], rows0_v, sem0)
    pltpu.async_copy(y_hbm.at[src_v.at[1]], rows1_v, sem1)

    def half(j, rows_v, sem):
        pltpu.make_async_copy(y_hbm.at[src_v.at[0]], rows_v, sem).wait()
        pltpu.sync_copy(rows_v, acc_sp.at[dst_v.at[j]], add=True)
        if with_deg:
            pltpu.sync_copy(ones_v, deg_sp.at[dst_v.at[j]], add=True)
        nxt = jnp.minimum(j + 2, _G - 1)
        pltpu.async_copy(y_hbm.at[src_v.at[nxt]], rows_v, sem)

    def step(p, _):
        half(2 * p, rows0_v, sem0)
        half(2 * p + 1, rows1_v, sem1)
        return 0

    lax.fori_loop(0, _G // 2, step, 0)
    pltpu.make_async_copy(y_hbm.at[src_v.at[0]], rows0_v, sem0).wait()
    pltpu.make_async_copy(y_hbm.at[src_v.at[0]], rows1_v, sem1).wait()
    plsc.subcore_barrier()

    pltpu.sync_copy(acc_sp.at[pl.ds(row0, _RPS)], acc_out.at[c, s])
    if with_deg:
        pltpu.sync_copy(deg_sp.at[pl.ds(row0, _RPS)], deg_out.at[c, s])


def _make_agg(with_deg):
    mesh = plsc.VectorSubcoreMesh(core_axis_name="c", subcore_axis_name="s")
    out_type = [jax.ShapeDtypeStruct((_NC, _NS, _RPS, _H), jnp.float32)]
    scratch = [
        pltpu.VMEM((_G, _B), jnp.int32),
        pltpu.VMEM((_G, _B), jnp.int32),
        pltpu.VMEM((_B, _H), jnp.float32),
        pltpu.VMEM((_B, _H), jnp.float32),
        pltpu.VMEM((_ZR, _H), jnp.float32),
        pltpu.VMEM_SHARED((_N, _H), jnp.float32),
        pltpu.SemaphoreType.DMA,
        pltpu.SemaphoreType.DMA,
    ]
    if with_deg:
        out_type.append(jax.ShapeDtypeStruct((_NC, _NS, _RPS, _DW), jnp.float32))
        scratch += [
            pltpu.VMEM((_B, _DW), jnp.float32),
            pltpu.VMEM((_ZR, _DW), jnp.float32),
            pltpu.VMEM_SHARED((_N, _DW), jnp.float32),
        ]
    return pl.kernel(
        functools.partial(_agg_body, with_deg),
        out_type=out_type,
        mesh=mesh,
        scratch_types=scratch,
        compiler_params=pltpu.CompilerParams(use_tc_tiling_on_sc=False),
    )


_agg_deg = _make_agg(True)
_agg = _make_agg(False)


def _proj1_body(x_ref, wl_ref, wr_ref, bl_ref, y_ref, z_ref):
    xb = x_ref[...]
    dn = (((1,), (1,)), ((), ()))
    y_ref[...] = lax.dot_general(xb, wl_ref[...], dn,
                                 preferred_element_type=jnp.float32)
    z_ref[...] = lax.dot_general(xb, wr_ref[...], dn,
                                 preferred_element_type=jnp.float32) + bl_ref[...]


def _proj1(x, wl1, wr1, bl1):
    bn = 2000
    return pl.pallas_call(
        _proj1_body,
        grid=(_N // bn,),
        in_specs=[
            pl.BlockSpec((bn, _D), lambda i: (i, 0)),
            pl.BlockSpec((_H, _D), lambda i: (0, 0)),
            pl.BlockSpec((_H, _D), lambda i: (0, 0)),
            pl.BlockSpec((1, _H), lambda i: (0, 0)),
        ],
        out_specs=[
            pl.BlockSpec((bn, _H), lambda i: (i, 0)),
            pl.BlockSpec((bn, _H), lambda i: (i, 0)),
        ],
        out_shape=[
            jax.ShapeDtypeStruct((_N, _H), jnp.float32),
            jax.ShapeDtypeStruct((_N, _H), jnp.float32),
        ],
    )(x, wl1, wr1, bl1)


def _mid_body(s_ref, d_ref, z_ref, h_ref, dinv_ref):
    ssum = s_ref[0] + s_ref[1]
    deg = d_ref[0, :, 0:1] + d_ref[1, :, 0:1]
    dinv = 1.0 / jnp.maximum(deg, 1.0)
    h_ref[...] = jnp.maximum(ssum * dinv + z_ref[...], 0.0)
    dinv_ref[...] = dinv


def _mid(s1, deg, z1):
    bn = 2000
    return pl.pallas_call(
        _mid_body,
        grid=(_N // bn,),
        in_specs=[
            pl.BlockSpec((_NC, bn, _H), lambda i: (0, i, 0)),
            pl.BlockSpec((_NC, bn, _DW), lambda i: (0, i, 0)),
            pl.BlockSpec((bn, _H), lambda i: (i, 0)),
        ],
        out_specs=[
            pl.BlockSpec((bn, _H), lambda i: (i, 0)),
            pl.BlockSpec((bn, 1), lambda i: (i, 0)),
        ],
        out_shape=[
            jax.ShapeDtypeStruct((_N, _H), jnp.float32),
            jax.ShapeDtypeStruct((_N, 1), jnp.float32),
        ],
    )(s1, deg, z1)


def _final_body(s2_ref, dinv_ref, h_ref, wl2_ref, bl2_ref, wr2_ref, o_ref):
    mean2 = (s2_ref[0] + s2_ref[1]) * dinv_ref[...]
    dn = (((1,), (1,)), ((), ()))
    t = (lax.dot_general(mean2, wl2_ref[...], dn,
                         preferred_element_type=jnp.float32)
         + bl2_ref[...]
         + lax.dot_general(h_ref[...], wr2_ref[...], dn,
                           preferred_element_type=jnp.float32))
    m = jnp.max(t, axis=1, keepdims=True)
    lse = jnp.log(jnp.sum(jnp.exp(t - m), axis=1, keepdims=True))
    o_ref[...] = t - m - lse


def _final(s2, dinv, h, wl2, bl2, wr2):
    bn = 2000
    return pl.pallas_call(
        _final_body,
        grid=(_N // bn,),
        in_specs=[
            pl.BlockSpec((_NC, bn, _H), lambda i: (0, i, 0)),
            pl.BlockSpec((bn, 1), lambda i: (i, 0)),
            pl.BlockSpec((bn, _H), lambda i: (i, 0)),
            pl.BlockSpec((_C, _H), lambda i: (0, 0)),
            pl.BlockSpec((1, _C), lambda i: (0, 0)),
            pl.BlockSpec((_C, _H), lambda i: (0, 0)),
        ],
        out_specs=pl.BlockSpec((bn, _C), lambda i: (i, 0)),
        out_shape=jax.ShapeDtypeStruct((_N, _C), jnp.float32),
    )(s2, dinv, h, wl2, bl2, wr2)


def kernel(x, edge_index, Wl1, bl1, Wr1, Wl2, bl2, Wr2):
    src = edge_index[0].reshape(_NW, _G, _B)
    dst = edge_index[1].reshape(_NW, _G, _B)
    y1, z1 = _proj1(x, Wl1, Wr1, bl1.reshape(1, _H))
    s1, deg = _agg_deg(y1, src, dst)
    s1 = s1.reshape(_NC, _N, _H)
    deg = deg.reshape(_NC, _N, _DW)
    h, dinv = _mid(s1, deg, z1)
    s2 = _agg(h, src, dst)
    if isinstance(s2, (list, tuple)):
        s2 = s2[0]
    s2 = s2.reshape(_NC, _N, _H)
    out = _final(s2, dinv, h, Wl2, bl2.reshape(1, _C), Wr2)
    return out


# speedup vs baseline: 20.2309x; 1.2195x over previous
import functools

import jax
import jax.numpy as jnp
from jax import lax
from jax.experimental import pallas as pl
from jax.experimental.pallas import tpu as pltpu
from jax.experimental.pallas import tpu_sc as plsc

_N = 10000
_E = 320000
_D = 128
_H = 32
_C = 40

_NC = 2
_NS = 16
_NW = _NC * _NS
_EPW = _E // _NW
_B = 125
_G = _EPW // _B
_NB = 4
_RPS = _N // _NS
_ZR = 125
_DW = 16


def _agg_body(with_deg, y_hbm, src_hbm, dst_hbm, *refs):
    if with_deg:
        (acc_out, deg_out, src_v, dst_v, rows_v, zb_v, acc_sp,
         gsems, ssems, ones_v, zd_v, deg_sp, dsems) = refs
    else:
        (acc_out, src_v, dst_v, rows_v, zb_v, acc_sp, gsems, ssems) = refs
    c = lax.axis_index("c")
    s = lax.axis_index("s")
    wid = c * _NS + s

    pltpu.sync_copy(src_hbm.at[wid], src_v)
    pltpu.sync_copy(dst_hbm.at[wid], dst_v)

    def zf(i, _):
        zb_v[i, pl.ds(0, 16)] = jnp.zeros((16,), jnp.float32)
        zb_v[i, pl.ds(16, 16)] = jnp.zeros((16,), jnp.float32)
        return 0

    lax.fori_loop(0, _ZR, zf, 0)
    row0 = s * _RPS
    for k in range(_RPS // _ZR):
        pltpu.sync_copy(zb_v, acc_sp.at[pl.ds(row0 + k * _ZR, _ZR)])
    if with_deg:
        def zf2(i, _):
            zd_v[i, :] = jnp.zeros((_DW,), jnp.float32)
            return 0

        lax.fori_loop(0, _ZR, zf2, 0)

        def of(i, _):
            ones_v[i, :] = jnp.ones((_DW,), jnp.float32)
            return 0

        lax.fori_loop(0, _B, of, 0)
        for k in range(_RPS // _ZR):
            pltpu.sync_copy(zd_v, deg_sp.at[pl.ds(row0 + k * _ZR, _ZR)])

    plsc.subcore_barrier()

    for b in range(_NB):
        pltpu.async_copy(y_hbm.at[src_v.at[b]], rows_v.at[b], gsems.at[b])

    def step(p, _):
        for b in range(_NB):
            j = _NB * p + b
            pltpu.make_async_copy(
                y_hbm.at[src_v.at[0]], rows_v.at[b], gsems.at[b]).wait()
            pltpu.async_copy(
                rows_v.at[b], acc_sp.at[dst_v.at[j]], ssems.at[b], add=True)
            if with_deg:
                pltpu.async_copy(
                    ones_v, deg_sp.at[dst_v.at[j]], dsems.at[b], add=True)
        for b in range(_NB):
            j = _NB * p + b
            pltpu.make_async_copy(
                rows_v.at[b], acc_sp.at[pl.ds(0, _B)], ssems.at[b]).wait()
            if with_deg:
                pltpu.make_async_copy(
                    ones_v, deg_sp.at[pl.ds(0, _B)], dsems.at[b]).wait()
            nxt = jnp.minimum(j + _NB, _G - 1)
            pltpu.async_copy(y_hbm.at[src_v.at[nxt]], rows_v.at[b], gsems.at[b])
        return 0

    lax.fori_loop(0, _G // _NB, step, 0)
    for b in range(_NB):
        pltpu.make_async_copy(
            y_hbm.at[src_v.at[0]], rows_v.at[b], gsems.at[b]).wait()
    plsc.subcore_barrier()

    pltpu.sync_copy(acc_sp.at[pl.ds(row0, _RPS)], acc_out.at[c, s])
    if with_deg:
        pltpu.sync_copy(deg_sp.at[pl.ds(row0, _RPS)], deg_out.at[c, s])


def _make_agg(with_deg):
    mesh = plsc.VectorSubcoreMesh(core_axis_name="c", subcore_axis_name="s")
    out_type = [jax.ShapeDtypeStruct((_NC, _NS, _RPS, _H), jnp.float32)]
    scratch = [
        pltpu.VMEM((_G, _B), jnp.int32),
        pltpu.VMEM((_G, _B), jnp.int32),
        pltpu.VMEM((_NB, _B, _H), jnp.float32),
        pltpu.VMEM((_ZR, _H), jnp.float32),
        pltpu.VMEM_SHARED((_N, _H), jnp.float32),
        pltpu.SemaphoreType.DMA((_NB,)),
        pltpu.SemaphoreType.DMA((_NB,)),
    ]
    if with_deg:
        out_type.append(jax.ShapeDtypeStruct((_NC, _NS, _RPS, _DW), jnp.float32))
        scratch += [
            pltpu.VMEM((_B, _DW), jnp.float32),
            pltpu.VMEM((_ZR, _DW), jnp.float32),
            pltpu.VMEM_SHARED((_N, _DW), jnp.float32),
            pltpu.SemaphoreType.DMA((_NB,)),
        ]
    return pl.kernel(
        functools.partial(_agg_body, with_deg),
        out_type=out_type,
        mesh=mesh,
        scratch_types=scratch,
        compiler_params=pltpu.CompilerParams(use_tc_tiling_on_sc=False),
    )


_agg_deg = _make_agg(True)
_agg = _make_agg(False)


def _proj1_body(x_ref, wl_ref, wr_ref, bl_ref, y_ref, z_ref):
    xb = x_ref[...]
    dn = (((1,), (1,)), ((), ()))
    y_ref[...] = lax.dot_general(xb, wl_ref[...], dn,
                                 preferred_element_type=jnp.float32)
    z_ref[...] = lax.dot_general(xb, wr_ref[...], dn,
                                 preferred_element_type=jnp.float32) + bl_ref[...]


def _proj1(x, wl1, wr1, bl1):
    bn = 2000
    return pl.pallas_call(
        _proj1_body,
        grid=(_N // bn,),
        in_specs=[
            pl.BlockSpec((bn, _D), lambda i: (i, 0)),
            pl.BlockSpec((_H, _D), lambda i: (0, 0)),
            pl.BlockSpec((_H, _D), lambda i: (0, 0)),
            pl.BlockSpec((1, _H), lambda i: (0, 0)),
        ],
        out_specs=[
            pl.BlockSpec((bn, _H), lambda i: (i, 0)),
            pl.BlockSpec((bn, _H), lambda i: (i, 0)),
        ],
        out_shape=[
            jax.ShapeDtypeStruct((_N, _H), jnp.float32),
            jax.ShapeDtypeStruct((_N, _H), jnp.float32),
        ],
    )(x, wl1, wr1, bl1)


def _mid_body(s_ref, d_ref, z_ref, h_ref, dinv_ref):
    ssum = s_ref[0] + s_ref[1]
    deg = d_ref[0, :, 0:1] + d_ref[1, :, 0:1]
    dinv = 1.0 / jnp.maximum(deg, 1.0)
    h_ref[...] = jnp.maximum(ssum * dinv + z_ref[...], 0.0)
    dinv_ref[...] = dinv


def _mid(s1, deg, z1):
    bn = 2000
    return pl.pallas_call(
        _mid_body,
        grid=(_N // bn,),
        in_specs=[
            pl.BlockSpec((_NC, bn, _H), lambda i: (0, i, 0)),
            pl.BlockSpec((_NC, bn, _DW), lambda i: (0, i, 0)),
            pl.BlockSpec((bn, _H), lambda i: (i, 0)),
        ],
        out_specs=[
            pl.BlockSpec((bn, _H), lambda i: (i, 0)),
            pl.BlockSpec((bn, 1), lambda i: (i, 0)),
        ],
        out_shape=[
            jax.ShapeDtypeStruct((_N, _H), jnp.float32),
            jax.ShapeDtypeStruct((_N, 1), jnp.float32),
        ],
    )(s1, deg, z1)


def _final_body(s2_ref, dinv_ref, h_ref, wl2_ref, bl2_ref, wr2_ref, o_ref):
    mean2 = (s2_ref[0] + s2_ref[1]) * dinv_ref[...]
    dn = (((1,), (1,)), ((), ()))
    t = (lax.dot_general(mean2, wl2_ref[...], dn,
                         preferred_element_type=jnp.float32)
         + bl2_ref[...]
         + lax.dot_general(h_ref[...], wr2_ref[...], dn,
                           preferred_element_type=jnp.float32))
    m = jnp.max(t, axis=1, keepdims=True)
    lse = jnp.log(jnp.sum(jnp.exp(t - m), axis=1, keepdims=True))
    o_ref[...] = t - m - lse


def _final(s2, dinv, h, wl2, bl2, wr2):
    bn = 2000
    return pl.pallas_call(
        _final_body,
        grid=(_N // bn,),
        in_specs=[
            pl.BlockSpec((_NC, bn, _H), lambda i: (0, i, 0)),
            pl.BlockSpec((bn, 1), lambda i: (i, 0)),
            pl.BlockSpec((bn, _H), lambda i: (i, 0)),
            pl.BlockSpec((_C, _H), lambda i: (0, 0)),
            pl.BlockSpec((1, _C), lambda i: (0, 0)),
            pl.BlockSpec((_C, _H), lambda i: (0, 0)),
        ],
        out_specs=pl.BlockSpec((bn, _C), lambda i: (i, 0)),
        out_shape=jax.ShapeDtypeStruct((_N, _C), jnp.float32),
    )(s2, dinv, h, wl2, bl2, wr2)


def kernel(x, edge_index, Wl1, bl1, Wr1, Wl2, bl2, Wr2):
    src = edge_index[0].reshape(_NW, _G, _B)
    dst = edge_index[1].reshape(_NW, _G, _B)
    y1, z1 = _proj1(x, Wl1, Wr1, bl1.reshape(1, _H))
    s1, deg = _agg_deg(y1, src, dst)
    s1 = s1.reshape(_NC, _N, _H)
    deg = deg.reshape(_NC, _N, _DW)
    h, dinv = _mid(s1, deg, z1)
    s2 = _agg(h, src, dst)
    if isinstance(s2, (list, tuple)):
        s2 = s2[0]
    s2 = s2.reshape(_NC, _N, _H)
    out = _final(s2, dinv, h, Wl2, bl2.reshape(1, _C), Wr2)
    return out
